# baseline (speedup 1.0000x reference)
import functools

import jax
import jax.numpy as jnp
from jax import lax
from jax.experimental import pallas as pl
from jax.experimental.pallas import tpu as pltpu
from jax.experimental.pallas import tpu_sc as plsc

NC = 2
NS = 16
L = 16
NW = NC * NS

_P0 = 3.511021357038846e-06
_P1 = 0.9997923620654405
_P2 = -0.4969774307189262
_P3 = 0.31458917398845154
_P4 = -0.18878082354758763
_P5 = 0.08172564528980446
_P6 = -0.017207799230048133


def _log1p_poly(z):
    p = _P6
    p = p * z + _P5
    p = p * z + _P4
    p = p * z + _P3
    p = p * z + _P2
    p = p * z + _P1
    return p * z + _P0


def _sc_body(featT_hbm, mr_hbm, mc_hbm, params_hbm, out_hbm,
             featT_v, ur_v, uc_v, params_v, rows_v, stage_v):
    V = featT_hbm.shape[1]
    rows_per_w = V // NW
    chunk = rows_v.shape[0]
    nchunks = rows_per_w // chunk
    ncols = V // L

    cid = lax.axis_index("c")
    sid = lax.axis_index("s")
    wid = sid * NC + cid

    pltpu.sync_copy(featT_hbm, featT_v)
    pltpu.sync_copy(params_hbm, params_v)

    def lane_splat(k):
        return plsc.load_gather(params_v, [jnp.full((L,), k, jnp.int32)])

    dwr = [lane_splat(4 + k) - lane_splat(k) for k in range(4)]
    dbr = lane_splat(9) - lane_splat(8)
    dwc = [lane_splat(14 + k) - lane_splat(10 + k) for k in range(4)]
    dbc = lane_splat(19) - lane_splat(18)

    def build_u(dw, u_ref):
        def step(i, carry):
            sl = pl.ds(i * L, L)
            u_ref[sl] = (dw[0] * featT_v[0, sl] + dw[1] * featT_v[1, sl]
                         + dw[2] * featT_v[2, sl] + dw[3] * featT_v[3, sl])
            return carry
        lax.fori_loop(0, ncols, step, 0)

    build_u(dwr, ur_v)
    build_u(dwc, uc_v)

    zero = jnp.zeros((L,), jnp.float32)

    def one_matrix(m_hbm, u_ref, db, slot):
        def chunk_loop(k, carry):
            acc, tacc = carry
            base = wid * rows_per_w + k * chunk
            pltpu.sync_copy(m_hbm.at[pl.ds(base, chunk)], rows_v)
            for r in range(chunk):
                i = base + r
                ui = plsc.load_gather(u_ref, [jnp.full((L,), i, jnp.int32)]) + db

                def col_loop(cc, carry2):
                    a2, t2 = carry2
                    sl = pl.ds(cc * L, L)
                    mv = rows_v[r, sl]
                    uj = u_ref[sl]
                    d = ui - uj
                    t = jnp.where(mv != 0, 1.0, 0.0)
                    z = jnp.exp(-jnp.abs(d))
                    sp = jnp.maximum(d, 0.0) + _log1p_poly(z)
                    a2 = a2 + ((0.2 + 0.8 * t) * sp - t * d)
                    t2 = t2 + t
                    return (a2, t2)

                acc, tacc = lax.fori_loop(0, ncols, col_loop, (acc, tacc),
                                          unroll=4)
            return (acc, tacc)

        acc, tacc = lax.fori_loop(0, nchunks, chunk_loop, (zero, zero))
        stage_v[...] = acc
        pltpu.sync_copy(stage_v, out_hbm.at[slot, wid])
        stage_v[...] = tacc
        pltpu.sync_copy(stage_v, out_hbm.at[slot + 1, wid])

    one_matrix(mr_hbm, ur_v, dbr, 0)
    one_matrix(mc_hbm, uc_v, dbc, 2)


def _finalize_body(part_ref, out_ref):
    total = float(_V_STATIC) * float(_V_STATIC)
    p = part_ref[...]
    s0 = jnp.sum(p[0])
    s1 = jnp.sum(p[1])
    s2 = jnp.sum(p[2])
    s3 = jnp.sum(p[3])
    out_ref[0] = s0 / (0.2 * total + 0.8 * s1) + s2 / (0.2 * total + 0.8 * s3)


_V_STATIC = 2048


def kernel(data, row_matrix, col_matrix, num_vertices, Wr, br, Wc, bc):
    del num_vertices
    V = row_matrix.shape[1]
    feat = data[0, :, :4]
    featT = jnp.transpose(feat)
    mr = row_matrix[0]
    mc = col_matrix[0]
    params = jnp.concatenate(
        [Wr[:, 0], Wr[:, 1], br, Wc[:, 0], Wc[:, 1], bc,
         jnp.zeros((12,), jnp.float32)])

    mesh = plsc.VectorSubcoreMesh(core_axis_name="c", subcore_axis_name="s")
    sc_call = pl.kernel(
        _sc_body, mesh=mesh,
        out_type=jax.ShapeDtypeStruct((4, NW, L), jnp.float32),
        scratch_types=[
            pltpu.VMEM((4, V), jnp.float32),
            pltpu.VMEM((V,), jnp.float32),
            pltpu.VMEM((V,), jnp.float32),
            pltpu.VMEM((2 * L,), jnp.float32),
            pltpu.VMEM((8, V), jnp.int32),
            pltpu.VMEM((L,), jnp.float32),
        ],
        compiler_params=pltpu.CompilerParams(needs_layout_passes=False),
    )
    partials = sc_call(featT, mr, mc, params)

    out = pl.pallas_call(
        _finalize_body,
        in_specs=[pl.BlockSpec((4, NW, L), lambda: (0, 0, 0))],
        out_specs=pl.BlockSpec(memory_space=pltpu.SMEM),
        out_shape=jax.ShapeDtypeStruct((1,), jnp.float32),
    )(partials)
    return out


# speedup vs baseline: 1.1648x; 1.1648x over previous
import functools

import jax
import jax.numpy as jnp
from jax import lax
from jax.experimental import pallas as pl
from jax.experimental.pallas import tpu as pltpu
from jax.experimental.pallas import tpu_sc as plsc

NC = 2
NS = 16
L = 16
NW = NC * NS

_P0 = 0.00014158017492749142
_P1 = 0.9954266617754249
_P2 = -0.4640707011025748
_P3 = 0.21640858368174304
_P4 = -0.05486231128931281


def _log1p_poly(z):
    p = _P4
    p = p * z + _P3
    p = p * z + _P2
    p = p * z + _P1
    return p * z + _P0


def _sc_body(featT_hbm, mr_hbm, mc_hbm, params_hbm, out_hbm,
             featT_v, ur_v, uc_v, params_v, rows_v, stage_v):
    V = featT_hbm.shape[1]
    rows_per_w = V // NW
    chunk = rows_v.shape[0]
    nchunks = rows_per_w // chunk
    ncols = V // L

    cid = lax.axis_index("c")
    sid = lax.axis_index("s")
    wid = sid * NC + cid

    pltpu.sync_copy(featT_hbm, featT_v)
    pltpu.sync_copy(params_hbm, params_v)

    def lane_splat(k):
        return plsc.load_gather(params_v, [jnp.full((L,), k, jnp.int32)])

    dwr = [lane_splat(4 + k) - lane_splat(k) for k in range(4)]
    dbr = lane_splat(9) - lane_splat(8)
    dwc = [lane_splat(14 + k) - lane_splat(10 + k) for k in range(4)]
    dbc = lane_splat(19) - lane_splat(18)

    def build_u(dw, u_ref):
        def step(i, carry):
            sl = pl.ds(i * L, L)
            u_ref[sl] = (dw[0] * featT_v[0, sl] + dw[1] * featT_v[1, sl]
                         + dw[2] * featT_v[2, sl] + dw[3] * featT_v[3, sl])
            return carry
        lax.fori_loop(0, ncols, step, 0)

    build_u(dwr, ur_v)
    build_u(dwc, uc_v)

    zero = jnp.zeros((L,), jnp.float32)

    def one_matrix(m_hbm, u_ref, db, slot):
        def chunk_loop(k, carry):
            acc, wacc = carry
            base = wid * rows_per_w + k * chunk
            pltpu.sync_copy(m_hbm.at[pl.ds(base, chunk)], rows_v)
            for r in range(chunk):
                i = base + r
                ui = plsc.load_gather(u_ref, [jnp.full((L,), i, jnp.int32)]) + db

                def col_loop(cc, carry2):
                    a2, w2 = carry2
                    sl = pl.ds(cc * L, L)
                    mv = rows_v[r, sl]
                    uj = u_ref[sl]
                    d = ui - uj
                    nz = mv != 0
                    arg = jnp.where(nz, -d, d)
                    z = jnp.exp(-jnp.abs(d))
                    nll = jnp.maximum(arg, 0.0) + _log1p_poly(z)
                    w = jnp.where(nz, 1.0, 0.2)
                    a2 = a2 + w * nll
                    w2 = w2 + w
                    return (a2, w2)

                acc, wacc = lax.fori_loop(0, ncols, col_loop, (acc, wacc),
                                          unroll=8)
            return (acc, wacc)

        acc, wacc = lax.fori_loop(0, nchunks, chunk_loop, (zero, zero))
        stage_v[...] = acc
        pltpu.sync_copy(stage_v, out_hbm.at[slot, wid])
        stage_v[...] = wacc
        pltpu.sync_copy(stage_v, out_hbm.at[slot + 1, wid])

    one_matrix(mr_hbm, ur_v, dbr, 0)
    one_matrix(mc_hbm, uc_v, dbc, 2)


def _finalize_body(part_ref, out_ref):
    p = part_ref[...]
    s0 = jnp.sum(p[0])
    s1 = jnp.sum(p[1])
    s2 = jnp.sum(p[2])
    s3 = jnp.sum(p[3])
    out_ref[0] = s0 / s1 + s2 / s3


def kernel(data, row_matrix, col_matrix, num_vertices, Wr, br, Wc, bc):
    del num_vertices
    V = row_matrix.shape[1]
    feat = data[0, :, :4]
    featT = jnp.transpose(feat)
    mr = row_matrix[0]
    mc = col_matrix[0]
    params = jnp.concatenate(
        [Wr[:, 0], Wr[:, 1], br, Wc[:, 0], Wc[:, 1], bc,
         jnp.zeros((12,), jnp.float32)])

    mesh = plsc.VectorSubcoreMesh(core_axis_name="c", subcore_axis_name="s")
    sc_call = pl.kernel(
        _sc_body, mesh=mesh,
        out_type=jax.ShapeDtypeStruct((4, NW, L), jnp.float32),
        scratch_types=[
            pltpu.VMEM((4, V), jnp.float32),
            pltpu.VMEM((V,), jnp.float32),
            pltpu.VMEM((V,), jnp.float32),
            pltpu.VMEM((2 * L,), jnp.float32),
            pltpu.VMEM((8, V), jnp.int32),
            pltpu.VMEM((L,), jnp.float32),
        ],
        compiler_params=pltpu.CompilerParams(needs_layout_passes=False),
    )
    partials = sc_call(featT, mr, mc, params)

    out = pl.pallas_call(
        _finalize_body,
        in_specs=[pl.BlockSpec((4, NW, L), lambda: (0, 0, 0))],
        out_specs=pl.BlockSpec(memory_space=pltpu.SMEM),
        out_shape=jax.ShapeDtypeStruct((1,), jnp.float32),
    )(partials)
    return out
